# baseline (speedup 1.0000x reference)
import jax
import jax.numpy as jnp
from jax import lax
from jax.experimental import pallas as pl
from jax.experimental.pallas import tpu as pltpu
from jax.experimental.pallas import tpu_sc as plsc

_K = 1000
_D = 128
_B = 4096
_L = 200

_NC = 2
_NS = 16
_NW = _NC * _NS

_N = _B * _L
_PER_W = _N // _NW
_SUB = 128
_NSUB = _PER_W // _SUB
_NBUF = 4

_ROWS_MAIN = 64
_ROWS_LAST = _K - (_NS - 1) * _ROWS_MAIN


def _stage_rows(rows, base_row, e_hbm, l_hbm, h_hbm, r16_hbm, table_sh,
                ebuf, lh_v, r_v):
    pltpu.sync_copy(e_hbm.at[pl.ds(base_row, rows)], ebuf.at[pl.ds(0, rows)])
    pltpu.sync_copy(r16_hbm.at[pl.ds(base_row, rows)], r_v.at[pl.ds(0, rows)])
    pltpu.sync_copy(l_hbm, lh_v.at[0])
    pltpu.sync_copy(h_hbm, lh_v.at[1])
    lvs = [lh_v[0, pl.ds(16 * j, 16)] for j in range(_D // 16)]
    hvs = [lh_v[1, pl.ds(16 * j, 16)] for j in range(_D // 16)]

    def row(k, carry):
        rk = r_v[k, :]
        for j in range(_D // 16):
            s = pl.ds(16 * j, 16)
            ebuf[k, s] = rk * lvs[j] + (1.0 - rk) * hvs[j] + ebuf[k, s]
        return carry

    lax.fori_loop(0, rows, row, 0)
    pltpu.sync_copy(ebuf.at[pl.ds(0, rows)], table_sh.at[pl.ds(base_row, rows)])


def _embed_body(e_hbm, l_hbm, h_hbm, r16_hbm, idx_hbm, out_hbm,
                table_sh, idx_v, rows_v, ebuf, lh_v, r_v, semg, semo):
    sid = lax.axis_index("s")
    wid = sid * _NC + lax.axis_index("c")
    base = wid * _NSUB

    pltpu.sync_copy(idx_hbm.at[pl.ds(base, _NSUB)], idx_v)

    @pl.when(sid < _NS - 1)
    def _():
        _stage_rows(_ROWS_MAIN, sid * _ROWS_MAIN, e_hbm, l_hbm, h_hbm,
                    r16_hbm, table_sh, ebuf, lh_v, r_v)

    @pl.when(sid == _NS - 1)
    def _():
        _stage_rows(_ROWS_LAST, (_NS - 1) * _ROWS_MAIN, e_hbm, l_hbm, h_hbm,
                    r16_hbm, table_sh, ebuf, lh_v, r_v)

    plsc.subcore_barrier()

    def group(p, carry):
        gathers = []
        for b in range(_NBUF):
            c = p * _NBUF + b

            @pl.when(p > 0)
            def _(b=b):
                pltpu.make_async_copy(
                    rows_v.at[b], out_hbm.at[pl.ds(0, _SUB)], semo.at[b]
                ).wait()

            gathers.append(
                pltpu.async_copy(table_sh.at[idx_v.at[c]], rows_v.at[b], semg)
            )
        for b in range(_NBUF):
            c = p * _NBUF + b
            gathers[b].wait()
            pltpu.async_copy(
                rows_v.at[b],
                out_hbm.at[pl.ds((base + c) * _SUB, _SUB)],
                semo.at[b],
            )
        return carry

    lax.fori_loop(0, _NSUB // _NBUF, group, 0)

    for b in range(_NBUF):
        pltpu.make_async_copy(
            rows_v.at[b], out_hbm.at[pl.ds(0, _SUB)], semo.at[b]
        ).wait()


@jax.jit
def kernel(idx, E, l, h, r):
    idx2 = idx.reshape(_N // _SUB, _SUB).astype(jnp.int32)

    embed = pl.kernel(
        _embed_body,
        out_type=jax.ShapeDtypeStruct((_N, _D), jnp.float32),
        mesh=plsc.VectorSubcoreMesh(
            core_axis_name="c", subcore_axis_name="s",
            num_cores=_NC, num_subcores=_NS,
        ),
        scratch_types=[
            pltpu.VMEM_SHARED((_K, _D), jnp.float32),
            pltpu.VMEM((_NSUB, _SUB), jnp.int32),
            pltpu.VMEM((_NBUF, _SUB, _D), jnp.float32),
            pltpu.VMEM((_ROWS_MAIN, _D), jnp.float32),
            pltpu.VMEM((2, _D), jnp.float32),
            pltpu.VMEM((_ROWS_MAIN, 16), jnp.float32),
            pltpu.SemaphoreType.DMA,
            pltpu.SemaphoreType.DMA((_NBUF,)),
        ],
    )
    r16 = jnp.broadcast_to(r.astype(jnp.float32)[:, None], (_K, 16))
    out = embed(E, l, h, r16, idx2)
    return out.reshape(_B, _L, _D)
